# baseline (speedup 1.0000x reference)
import numpy as np
import jax
import jax.numpy as jnp
from jax import lax
from jax.experimental import pallas as pl
from jax.experimental.pallas import tpu as pltpu
from jax.experimental.pallas import tpu_sc as plsc

_VOCAB = 100000
_D = 1024
_PE_LEN = 2048
_BATCH = 4
_B = _BATCH * _PE_LEN
_NC, _NS, _L = 2, 16, 16
_NW = _NC * _NS
_PER_W = _B // _NW
_C = 32
_NCHUNK = _PER_W // _C
_VPR = _D // _L
_SCALE = float(np.sqrt(float(_D)))


def _pe_table() -> np.ndarray:
    depth_h = _D / 2
    positions = np.arange(_PE_LEN)[:, np.newaxis]
    depths = np.arange(depth_h)[np.newaxis, :] / depth_h
    angle_rads = positions * (1 / 10000**depths)
    return np.concatenate(
        [np.sin(angle_rads), np.cos(angle_rads)], axis=-1
    ).astype(np.float32)


_PE = _pe_table()


def _sc_body(table_hbm, idx_hbm, pe_hbm, out_hbm, idx_v, rows_v, pe_v, sem):
    wid = lax.axis_index("s") * _NC + lax.axis_index("c")
    base = wid * _PER_W
    t0 = base % _PE_LEN

    pltpu.sync_copy(idx_hbm.at[pl.ds(base, _PER_W)], idx_v)

    def chunk_body(c, carry):
        row0 = c * _C
        pltpu.async_copy(
            table_hbm.at[idx_v.at[pl.ds(row0, _C)]], rows_v, sem
        ).wait()
        pltpu.sync_copy(pe_hbm.at[pl.ds(t0 + row0, _C)], pe_v)

        def vec_body(k, carry2):
            i = k // _VPR
            j = (k % _VPR) * _L
            rows_v[i, pl.ds(j, _L)] = (
                rows_v[i, pl.ds(j, _L)] * _SCALE + pe_v[i, pl.ds(j, _L)]
            )
            return carry2

        lax.fori_loop(0, _C * _VPR, vec_body, 0)
        pltpu.sync_copy(rows_v, out_hbm.at[pl.ds(base + row0, _C)])
        return carry

    lax.fori_loop(0, _NCHUNK, chunk_body, 0)


_sc_fn = pl.kernel(
    _sc_body,
    out_type=jax.ShapeDtypeStruct((_B, _D), jnp.float32),
    mesh=plsc.VectorSubcoreMesh(core_axis_name="c", subcore_axis_name="s"),
    scratch_types=[
        pltpu.VMEM((_PER_W,), jnp.int32),
        pltpu.VMEM((_C, _D), jnp.float32),
        pltpu.VMEM((_C, _D), jnp.float32),
        pltpu.SemaphoreType.DMA,
    ],
)


@jax.jit
def kernel(x, table):
    idx = x.reshape(_B)
    pe = jnp.asarray(_PE)
    out = _sc_fn(table, idx, pe)
    return out.reshape(_BATCH, _PE_LEN, _D)


# speedup vs baseline: 2.7064x; 2.7064x over previous
import numpy as np
import jax
import jax.numpy as jnp
from jax import lax
from jax.experimental import pallas as pl
from jax.experimental.pallas import tpu as pltpu
from jax.experimental.pallas import tpu_sc as plsc

_VOCAB = 100000
_D = 1024
_PE_LEN = 2048
_BATCH = 4
_B = _BATCH * _PE_LEN
_NC, _NS, _L = 2, 16, 16
_NW = _NC * _NS
_TW = _PE_LEN // _NW
_CH = 16
_NTC = _TW // _CH
_NSTEP = _NTC * _BATCH
_VPR = _D // _L
_SCALE = float(np.sqrt(float(_D)))


def _pe_table() -> np.ndarray:
    depth_h = _D / 2
    positions = np.arange(_PE_LEN)[:, np.newaxis]
    depths = np.arange(depth_h)[np.newaxis, :] / depth_h
    angle_rads = positions * (1 / 10000**depths)
    return np.concatenate(
        [np.sin(angle_rads), np.cos(angle_rads)], axis=-1
    ).astype(np.float32)


_PE = _pe_table()


def _sc_body(table_hbm, idx_hbm, pe_hbm, out_hbm,
             idx_v, pe_v, rows_v, out_v, gsem, osem):
    wid = lax.axis_index("s") * _NC + lax.axis_index("c")
    t0 = wid * _TW

    for b in range(_BATCH):
        pltpu.sync_copy(idx_hbm.at[pl.ds(b * _PE_LEN + t0, _TW)],
                        idx_v.at[pl.ds(b * _TW, _TW)])

    def gather_start(s):
        tc, b = divmod(s, _BATCH)
        return pltpu.async_copy(
            table_hbm.at[idx_v.at[pl.ds(b * _TW + tc * _CH, _CH)]],
            rows_v.at[s % 2], gsem.at[s % 2])

    g = [gather_start(0), None]
    st = [None, None]
    for tc in range(_NTC):
        pltpu.sync_copy(pe_hbm.at[pl.ds(t0 + tc * _CH, _CH)], pe_v)
        for b in range(_BATCH):
            s = tc * _BATCH + b
            if s + 1 < _NSTEP:
                g[(s + 1) % 2] = gather_start(s + 1)
            g[s % 2].wait()
            if st[s % 2] is not None:
                st[s % 2].wait()
            rbuf = rows_v.at[s % 2]
            obuf = out_v.at[s % 2]

            @plsc.parallel_loop(0, _CH * _VPR, unroll=8)
            def _compute(k, rbuf=rbuf, obuf=obuf):
                i = k // _VPR
                j = (k % _VPR) * _L
                obuf[i, pl.ds(j, _L)] = (
                    rbuf[i, pl.ds(j, _L)] * _SCALE + pe_v[i, pl.ds(j, _L)]
                )

            st[s % 2] = pltpu.async_copy(
                obuf,
                out_hbm.at[pl.ds(b * _PE_LEN + t0 + tc * _CH, _CH)],
                osem.at[s % 2])
    st[0].wait()
    st[1].wait()


_sc_fn = pl.kernel(
    _sc_body,
    out_type=jax.ShapeDtypeStruct((_B, _D), jnp.float32),
    mesh=plsc.VectorSubcoreMesh(core_axis_name="c", subcore_axis_name="s"),
    scratch_types=[
        pltpu.VMEM((_BATCH * _TW,), jnp.int32),
        pltpu.VMEM((_CH, _D), jnp.float32),
        pltpu.VMEM((2, _CH, _D), jnp.float32),
        pltpu.VMEM((2, _CH, _D), jnp.float32),
        pltpu.SemaphoreType.DMA((2,)),
        pltpu.SemaphoreType.DMA((2,)),
    ],
)


@jax.jit
def kernel(x, table):
    idx = x.reshape(_B)
    pe = jnp.asarray(_PE)
    out = _sc_fn(table, idx, pe)
    return out.reshape(_BATCH, _PE_LEN, _D)


# speedup vs baseline: 2.7847x; 1.0289x over previous
import numpy as np
import jax
import jax.numpy as jnp
from jax import lax
from jax.experimental import pallas as pl
from jax.experimental.pallas import tpu as pltpu
from jax.experimental.pallas import tpu_sc as plsc

_VOCAB = 100000
_D = 1024
_PE_LEN = 2048
_BATCH = 4
_B = _BATCH * _PE_LEN
_NC, _NS, _L = 2, 16, 16
_NW = _NC * _NS
_TW = _PE_LEN // _NW
_CH = 8
_NSTEP = _TW // _CH
_RPS = _BATCH * _CH
_VPR = _D // _L
_SCALE = float(np.sqrt(float(_D)))


def _pe_table() -> np.ndarray:
    depth_h = _D / 2
    positions = np.arange(_PE_LEN)[:, np.newaxis]
    depths = np.arange(depth_h)[np.newaxis, :] / depth_h
    angle_rads = positions * (1 / 10000**depths)
    return np.concatenate(
        [np.sin(angle_rads), np.cos(angle_rads)], axis=-1
    ).astype(np.float32)


_PE = _pe_table()


def _sc_body(table_hbm, idx_hbm, pe_hbm, out_hbm,
             idx_v, pe_v, rows_v, gsem, osem):
    wid = lax.axis_index("s") * _NC + lax.axis_index("c")
    t0 = wid * _TW

    pltpu.sync_copy(idx_hbm.at[pl.ds(wid * _NSTEP * _RPS, _NSTEP * _RPS)],
                    idx_v)

    def gather_start(tc):
        return pltpu.async_copy(
            table_hbm.at[idx_v.at[pl.ds(tc * _RPS, _RPS)]],
            rows_v.at[tc % 2], gsem.at[tc % 2])

    g = [gather_start(0), None]
    st = [None, None]
    for tc in range(_NSTEP):
        p = tc % 2
        q = (tc + 1) % 2
        pltpu.sync_copy(pe_hbm.at[pl.ds(t0 + tc * _CH, _CH)], pe_v)
        g[p].wait()
        if st[q] is not None:
            for d in st[q]:
                d.wait()
        if tc + 1 < _NSTEP:
            g[q] = gather_start(tc + 1)

        @plsc.parallel_loop(0, _CH * _VPR, unroll=2)
        def _compute(k, p=p):
            i = k // _VPR
            j = (k % _VPR) * _L
            pev = pe_v[i, pl.ds(j, _L)]
            for b in range(_BATCH):
                rows_v[p, b * _CH + i, pl.ds(j, _L)] = (
                    rows_v[p, b * _CH + i, pl.ds(j, _L)] * _SCALE + pev
                )

        st[p] = [
            pltpu.async_copy(
                rows_v.at[p, pl.ds(b * _CH, _CH)],
                out_hbm.at[pl.ds(b * _PE_LEN + t0 + tc * _CH, _CH)],
                osem.at[p])
            for b in range(_BATCH)
        ]
    for d in st[(_NSTEP - 1) % 2]:
        d.wait()


_sc_fn = pl.kernel(
    _sc_body,
    out_type=jax.ShapeDtypeStruct((_B, _D), jnp.float32),
    mesh=plsc.VectorSubcoreMesh(core_axis_name="c", subcore_axis_name="s"),
    scratch_types=[
        pltpu.VMEM((_NSTEP * _RPS,), jnp.int32),
        pltpu.VMEM((_CH, _D), jnp.float32),
        pltpu.VMEM((2, _RPS, _D), jnp.float32),
        pltpu.SemaphoreType.DMA((2,)),
        pltpu.SemaphoreType.DMA((2,)),
    ],
)


@jax.jit
def kernel(x, table):
    idx = (x.reshape(_BATCH, _NW, _NSTEP, _CH)
             .transpose(1, 2, 0, 3)
             .reshape(_B))
    pe = jnp.asarray(_PE)
    out = _sc_fn(table, idx, pe)
    return out.reshape(_BATCH, _PE_LEN, _D)


# speedup vs baseline: 2.8306x; 1.0165x over previous
import numpy as np
import jax
import jax.numpy as jnp
from jax import lax
from jax.experimental import pallas as pl
from jax.experimental.pallas import tpu as pltpu
from jax.experimental.pallas import tpu_sc as plsc

_VOCAB = 100000
_D = 1024
_PE_LEN = 2048
_BATCH = 4
_B = _BATCH * _PE_LEN
_NC, _NS, _L = 2, 16, 16
_NW = _NC * _NS
_TW = _PE_LEN // _NW
_CH = 8
_NSTEP = _TW // _CH
_RPS = _BATCH * _CH
_VPR = _D // _L
_SCALE = float(np.sqrt(float(_D)))


def _pe_table() -> np.ndarray:
    depth_h = _D / 2
    positions = np.arange(_PE_LEN)[:, np.newaxis]
    depths = np.arange(depth_h)[np.newaxis, :] / depth_h
    angle_rads = positions * (1 / 10000**depths)
    return np.concatenate(
        [np.sin(angle_rads), np.cos(angle_rads)], axis=-1
    ).astype(np.float32)


_PE = _pe_table()


def _sc_body(table_hbm, idx_hbm, pe_hbm, out_hbm,
             idx_v, pe_v, rows_v, gsem, psem, osem):
    wid = lax.axis_index("s") * _NC + lax.axis_index("c")
    t0 = wid * _TW

    pltpu.sync_copy(idx_hbm.at[pl.ds(wid * _NSTEP * _RPS, _NSTEP * _RPS)],
                    idx_v)

    def gather_start(tc):
        return pltpu.async_copy(
            table_hbm.at[idx_v.at[pl.ds(tc * _RPS, _RPS)]],
            rows_v.at[tc % 3], gsem.at[tc % 3])

    def pe_start(tc):
        return pltpu.async_copy(
            pe_hbm.at[pl.ds(t0 + tc * _CH, _CH)],
            pe_v.at[tc % 2], psem.at[tc % 2])

    g = [gather_start(0), None, None]
    pe_cp = [pe_start(0), None]
    st = [None, None, None]
    for tc in range(_NSTEP):
        p = tc % 3
        pp = tc % 2
        if tc + 1 < _NSTEP:
            pe_cp[(tc + 1) % 2] = pe_start(tc + 1)
        g[p].wait()
        pe_cp[pp].wait()
        if st[(tc + 1) % 3] is not None:
            for d in st[(tc + 1) % 3]:
                d.wait()
        if tc + 1 < _NSTEP:
            g[(tc + 1) % 3] = gather_start(tc + 1)

        @plsc.parallel_loop(0, _CH * _VPR, unroll=2)
        def _compute(k, p=p, pp=pp):
            i = k // _VPR
            j = (k % _VPR) * _L
            pev = pe_v[pp, i, pl.ds(j, _L)]
            for b in range(_BATCH):
                rows_v[p, b * _CH + i, pl.ds(j, _L)] = (
                    rows_v[p, b * _CH + i, pl.ds(j, _L)] * _SCALE + pev
                )

        st[p] = [
            pltpu.async_copy(
                rows_v.at[p, pl.ds(b * _CH, _CH)],
                out_hbm.at[pl.ds(b * _PE_LEN + t0 + tc * _CH, _CH)],
                osem.at[p])
            for b in range(_BATCH)
        ]
    for k in (_NSTEP - 2, _NSTEP - 1):
        for d in st[k % 3]:
            d.wait()


_sc_fn = pl.kernel(
    _sc_body,
    out_type=jax.ShapeDtypeStruct((_B, _D), jnp.float32),
    mesh=plsc.VectorSubcoreMesh(core_axis_name="c", subcore_axis_name="s"),
    scratch_types=[
        pltpu.VMEM((_NSTEP * _RPS,), jnp.int32),
        pltpu.VMEM((2, _CH, _D), jnp.float32),
        pltpu.VMEM((3, _RPS, _D), jnp.float32),
        pltpu.SemaphoreType.DMA((3,)),
        pltpu.SemaphoreType.DMA((2,)),
        pltpu.SemaphoreType.DMA((3,)),
    ],
)


@jax.jit
def kernel(x, table):
    idx = (x.reshape(_BATCH, _NW, _NSTEP, _CH)
             .transpose(1, 2, 0, 3)
             .reshape(_B))
    pe = jnp.asarray(_PE)
    out = _sc_fn(table, idx, pe)
    return out.reshape(_BATCH, _PE_LEN, _D)
